# baseline (speedup 1.0000x reference)
import functools

import jax
import jax.numpy as jnp
from jax import lax
from jax.experimental import pallas as pl
from jax.experimental.pallas import tpu as pltpu
from jax.experimental.pallas import tpu_sc as plsc

_B, _S, _D, _E, _K = 2, 2048, 1024, 8, 2
_N = _B * _S
_A = _N * _K
_TM = 256
_NPAD = _A + _E * _TM
_NT = _NPAD // _TM
_TT = 512
_NEG = -1e9


def _gating_body(x_ref, gw_ref, gb_ref, nz_ref, xn_ref, ti_ref, tp_ref,
                 us_ref, aux_ref):
    i = pl.program_id(0)
    x = x_ref[...]
    scores = jnp.mean(x, axis=-1, keepdims=True)
    sp = jnp.where(scores > 0.1, x, 0.0)
    xn = sp / (jnp.sum(sp, axis=-1, keepdims=True) + 1e-8)
    xn_ref[...] = xn

    logits = (jnp.dot(xn, gw_ref[...], preferred_element_type=jnp.float32)
              + gb_ref[...] + nz_ref[...])
    idx8 = lax.broadcasted_iota(jnp.int32, logits.shape, 1)
    m1 = jnp.max(logits, axis=-1, keepdims=True)
    i1 = jnp.min(jnp.where(logits == m1, idx8, _E), axis=-1, keepdims=True)
    rest = jnp.where(idx8 == i1, _NEG, logits)
    m2 = jnp.max(rest, axis=-1, keepdims=True)
    i2 = jnp.min(jnp.where(rest == m2, idx8, _E), axis=-1, keepdims=True)

    masked = jnp.where(logits >= m2, logits, _NEG)
    e = jnp.exp(masked - m1)
    p = e / jnp.sum(e, axis=-1, keepdims=True)
    p1 = jnp.sum(jnp.where(idx8 == i1, p, 0.0), axis=-1, keepdims=True)
    p2 = jnp.sum(jnp.where(idx8 == i2, p, 0.0), axis=-1, keepdims=True)
    ti_ref[...] = jnp.concatenate([i1, i2], axis=1)
    tp_ref[...] = jnp.concatenate([p1, p2], axis=1)

    @pl.when(i == 0)
    def _():
        us_ref[...] = jnp.zeros_like(us_ref)

    us_ref[...] += jnp.sum(p, axis=0, keepdims=True)

    @pl.when(i == pl.num_programs(0) - 1)
    def _():
        u = us_ref[0, :]
        imp = u / (jnp.sum(u) + 1e-10)
        mean = jnp.mean(imp)
        std = jnp.sqrt(jnp.mean((imp - mean) ** 2))
        aux_ref[...] = (std / (mean + 1e-10)).reshape(1, 1)


def _gating(xf, gate_w, gate_b, noise):
    nblk = _N // _TT
    return pl.pallas_call(
        _gating_body,
        grid=(nblk,),
        in_specs=[
            pl.BlockSpec((_TT, _D), lambda i: (i, 0)),
            pl.BlockSpec((_D, _E), lambda i: (0, 0)),
            pl.BlockSpec((1, _E), lambda i: (0, 0)),
            pl.BlockSpec((_TT, _E), lambda i: (i, 0)),
        ],
        out_specs=[
            pl.BlockSpec((_TT, _D), lambda i: (i, 0)),
            pl.BlockSpec((_TT, _K), lambda i: (i, 0)),
            pl.BlockSpec((_TT, _K), lambda i: (i, 0)),
            pl.BlockSpec((1, _E), lambda i: (0, 0)),
            pl.BlockSpec((1, 1), lambda i: (0, 0)),
        ],
        out_shape=[
            jax.ShapeDtypeStruct((_N, _D), jnp.float32),
            jax.ShapeDtypeStruct((_N, _K), jnp.int32),
            jax.ShapeDtypeStruct((_N, _K), jnp.float32),
            jax.ShapeDtypeStruct((1, _E), jnp.float32),
            jax.ShapeDtypeStruct((1, 1), jnp.float32),
        ],
    )(xf, gate_w, gate_b.reshape(1, _E), noise)


def _dispatch_meta(ti):
    e_flat = jnp.concatenate([ti[:, 0], ti[:, 1]])
    oh = (e_flat[:, None] == jnp.arange(_E)[None, :]).astype(jnp.int32)
    csum = jnp.cumsum(oh, axis=0)
    g = csum[-1]
    rank = jnp.take_along_axis(csum, e_flat[:, None], axis=1)[:, 0] - 1
    tiles = (g + _TM - 1) // _TM
    bounds = jnp.cumsum(tiles)
    off = (bounds - tiles) * _TM
    pos = off[e_flat] + rank
    spread = jnp.arange(_NPAD, dtype=jnp.int32) % _N
    tok2 = jnp.concatenate([jnp.arange(_N, dtype=jnp.int32)] * _K)
    tok_of_pos = spread.at[pos].set(tok2, unique_indices=True)
    tile_ids = jnp.arange(_NT, dtype=jnp.int32)
    eot = jnp.clip(
        jnp.searchsorted(bounds, tile_ids, side="right"),
        0, _E - 1).astype(jnp.int32)
    valid = (tile_ids < bounds[-1]).astype(jnp.int32)
    return tok_of_pos, pos, eot, valid


def _make_sc_gather(V, Brows, Dcols, ch):
    nw = 32
    b_per_w = Brows // nw
    assert Brows % (8 * nw) == 0 and b_per_w % ch == 0 and ch <= 128
    mesh = plsc.VectorSubcoreMesh(core_axis_name="c", subcore_axis_name="s")

    nck = b_per_w // ch

    @functools.partial(
        pl.kernel,
        out_type=jax.ShapeDtypeStruct((Brows, Dcols), jnp.float32),
        mesh=mesh,
        scratch_types=[
            pltpu.VMEM((b_per_w,), jnp.int32),
            pltpu.VMEM((ch, Dcols), jnp.float32),
            pltpu.VMEM((ch, Dcols), jnp.float32),
            pltpu.SemaphoreType.DMA,
            pltpu.SemaphoreType.DMA,
        ],
    )
    def gather(table_hbm, idx_hbm, out_hbm, idx_v, rows0, rows1, gsem, wsem):
        wid = lax.axis_index("s") * 2 + lax.axis_index("c")
        base = wid * b_per_w
        pltpu.sync_copy(idx_hbm.at[pl.ds(base, b_per_w)], idx_v)
        bufs = (rows0, rows1)

        g = [None] * nck
        w = [None] * nck
        g[0] = pltpu.async_copy(
            table_hbm.at[idx_v.at[pl.ds(0, ch)]], bufs[0], gsem)
        for j in range(nck):
            if j + 1 < nck:
                if j + 1 >= 2:
                    w[j - 1].wait()
                g[j + 1] = pltpu.async_copy(
                    table_hbm.at[idx_v.at[pl.ds((j + 1) * ch, ch)]],
                    bufs[(j + 1) % 2], gsem)
            g[j].wait()
            w[j] = pltpu.async_copy(
                bufs[j % 2], out_hbm.at[pl.ds(base + j * ch, ch)], wsem)
        if nck >= 2:
            w[nck - 2].wait()
        w[nck - 1].wait()

    return gather


def _ffn_body(eot_ref, valid_ref, xs_ref, w1_ref, b1_ref, w2_ref, b2_ref,
              out_ref):
    del eot_ref

    @pl.when(valid_ref[pl.program_id(0)] == 1)
    def _():
        x = xs_ref[...]
        h = (jnp.dot(x, w1_ref[0], preferred_element_type=jnp.float32,
                     precision=lax.Precision.DEFAULT)
             + b1_ref[0])
        h = h * jax.nn.sigmoid(h)
        out_ref[...] = (
            jnp.dot(h, w2_ref[0], preferred_element_type=jnp.float32,
                    precision=lax.Precision.DEFAULT)
            + b2_ref[0])


def _ffn(xs, w1, b1, w2, b2, eot, valid):
    grid_spec = pltpu.PrefetchScalarGridSpec(
        num_scalar_prefetch=2,
        grid=(_NT,),
        in_specs=[
            pl.BlockSpec((_TM, _D), lambda i, eot, v: (i, 0)),
            pl.BlockSpec((1, _D, 2 * _D), lambda i, eot, v: (eot[i], 0, 0)),
            pl.BlockSpec((1, 1, 2 * _D), lambda i, eot, v: (eot[i], 0, 0)),
            pl.BlockSpec((1, 2 * _D, _D), lambda i, eot, v: (eot[i], 0, 0)),
            pl.BlockSpec((1, 1, _D), lambda i, eot, v: (eot[i], 0, 0)),
        ],
        out_specs=pl.BlockSpec((_TM, _D), lambda i, eot, v: (i, 0)),
    )
    return pl.pallas_call(
        _ffn_body,
        grid_spec=grid_spec,
        out_shape=jax.ShapeDtypeStruct((_NPAD, _D), jnp.float32),
    )(eot, valid, xs, w1, b1.reshape(_E, 1, 2 * _D), w2,
      b2.reshape(_E, 1, _D))


def _combine_body(ga_ref, gb_ref, tp_ref, y_ref):
    y_ref[...] = (tp_ref[:, 0:1] * ga_ref[...]
                  + tp_ref[:, 1:2] * gb_ref[...])


def _combine(gpair, tp):
    nblk = _N // _TT
    return pl.pallas_call(
        _combine_body,
        grid=(nblk,),
        in_specs=[
            pl.BlockSpec((_TT, _D), lambda i: (i, 0)),
            pl.BlockSpec((_TT, _D), lambda i: (i + _N // _TT, 0)),
            pl.BlockSpec((_TT, _K), lambda i: (i, 0)),
        ],
        out_specs=pl.BlockSpec((_TT, _D), lambda i: (i, 0)),
        out_shape=jax.ShapeDtypeStruct((_N, _D), jnp.float32),
    )(gpair, gpair, tp)


def kernel(x, gate_w, gate_b, w1, b1, w2, b2):
    x = jnp.asarray(x, dtype=jnp.float32)
    xf = x.reshape(_N, _D)
    noise = (jax.random.normal(jax.random.key(42), (_B, _S, _E)) * 0.01
             ).reshape(_N, _E)

    xn, ti, tp, _us, aux = _gating(xf, gate_w, gate_b, noise)
    tok_of_pos, pos, eot, valid = _dispatch_meta(ti)

    xs = _make_sc_gather(_N, _NPAD, _D, 40)(xn, tok_of_pos)
    out2 = _ffn(xs, w1, b1, w2, b2, eot, valid)
    gpair = _make_sc_gather(_NPAD, _A, _D, 32)(out2, pos.astype(jnp.int32))
    y = _combine(gpair, tp)

    return (y.reshape(_B, _S, _D),
            ti.reshape(_B, _S, _K),
            aux[0, 0])


# speedup vs baseline: 1.0068x; 1.0068x over previous
import functools

import jax
import jax.numpy as jnp
from jax import lax
from jax.experimental import pallas as pl
from jax.experimental.pallas import tpu as pltpu
from jax.experimental.pallas import tpu_sc as plsc

_B, _S, _D, _E, _K = 2, 2048, 1024, 8, 2
_N = _B * _S
_A = _N * _K
_TM = 256
_NPAD = _A + _E * _TM
_NT = _NPAD // _TM
_TT = 512
_NEG = -1e9


def _gating_body(x_ref, gw_ref, gb_ref, nz_ref, xn_ref, ti_ref, tp_ref,
                 us_ref, aux_ref):
    i = pl.program_id(0)
    x = x_ref[...]
    scores = jnp.mean(x, axis=-1, keepdims=True)
    sp = jnp.where(scores > 0.1, x, 0.0)
    xn = sp / (jnp.sum(sp, axis=-1, keepdims=True) + 1e-8)
    xn_ref[...] = xn

    logits = (jnp.dot(xn, gw_ref[...], preferred_element_type=jnp.float32)
              + gb_ref[...] + nz_ref[...])
    idx8 = lax.broadcasted_iota(jnp.int32, logits.shape, 1)
    m1 = jnp.max(logits, axis=-1, keepdims=True)
    i1 = jnp.min(jnp.where(logits == m1, idx8, _E), axis=-1, keepdims=True)
    rest = jnp.where(idx8 == i1, _NEG, logits)
    m2 = jnp.max(rest, axis=-1, keepdims=True)
    i2 = jnp.min(jnp.where(rest == m2, idx8, _E), axis=-1, keepdims=True)

    masked = jnp.where(logits >= m2, logits, _NEG)
    e = jnp.exp(masked - m1)
    p = e / jnp.sum(e, axis=-1, keepdims=True)
    p1 = jnp.sum(jnp.where(idx8 == i1, p, 0.0), axis=-1, keepdims=True)
    p2 = jnp.sum(jnp.where(idx8 == i2, p, 0.0), axis=-1, keepdims=True)
    ti_ref[...] = jnp.concatenate([i1, i2], axis=1)
    tp_ref[...] = jnp.concatenate([p1, p2], axis=1)

    @pl.when(i == 0)
    def _():
        us_ref[...] = jnp.zeros_like(us_ref)

    us_ref[...] += jnp.sum(p, axis=0, keepdims=True)

    @pl.when(i == pl.num_programs(0) - 1)
    def _():
        u = us_ref[0, :]
        imp = u / (jnp.sum(u) + 1e-10)
        mean = jnp.mean(imp)
        std = jnp.sqrt(jnp.mean((imp - mean) ** 2))
        aux_ref[...] = (std / (mean + 1e-10)).reshape(1, 1)


def _gating(xf, gate_w, gate_b, noise):
    nblk = _N // _TT
    return pl.pallas_call(
        _gating_body,
        grid=(nblk,),
        in_specs=[
            pl.BlockSpec((_TT, _D), lambda i: (i, 0)),
            pl.BlockSpec((_D, _E), lambda i: (0, 0)),
            pl.BlockSpec((1, _E), lambda i: (0, 0)),
            pl.BlockSpec((_TT, _E), lambda i: (i, 0)),
        ],
        out_specs=[
            pl.BlockSpec((_TT, _D), lambda i: (i, 0)),
            pl.BlockSpec((_TT, _K), lambda i: (i, 0)),
            pl.BlockSpec((_TT, _K), lambda i: (i, 0)),
            pl.BlockSpec((1, _E), lambda i: (0, 0)),
            pl.BlockSpec((1, 1), lambda i: (0, 0)),
        ],
        out_shape=[
            jax.ShapeDtypeStruct((_N, _D), jnp.float32),
            jax.ShapeDtypeStruct((_N, _K), jnp.int32),
            jax.ShapeDtypeStruct((_N, _K), jnp.float32),
            jax.ShapeDtypeStruct((1, _E), jnp.float32),
            jax.ShapeDtypeStruct((1, 1), jnp.float32),
        ],
    )(xf, gate_w, gate_b.reshape(1, _E), noise)


def _dispatch_meta(ti):
    e_flat = jnp.concatenate([ti[:, 0], ti[:, 1]])
    oh = (e_flat[:, None] == jnp.arange(_E)[None, :]).astype(jnp.int32)
    csum = jnp.cumsum(oh, axis=0)
    g = csum[-1]
    rank = jnp.take_along_axis(csum, e_flat[:, None], axis=1)[:, 0] - 1
    tiles = (g + _TM - 1) // _TM
    bounds = jnp.cumsum(tiles)
    off = (bounds - tiles) * _TM
    pos = off[e_flat] + rank
    spread = jnp.arange(_NPAD, dtype=jnp.int32) % _N
    tok2 = jnp.concatenate([jnp.arange(_N, dtype=jnp.int32)] * _K)
    tok_of_pos = spread.at[pos].set(tok2)
    tile_ids = jnp.arange(_NT, dtype=jnp.int32)
    eot = jnp.clip(
        jnp.searchsorted(bounds, tile_ids, side="right"),
        0, _E - 1).astype(jnp.int32)
    valid = (tile_ids < bounds[-1]).astype(jnp.int32)
    return tok_of_pos, pos, eot, valid


def _make_sc_gather(V, Brows, Dcols, ch):
    nw = 32
    b_per_w = Brows // nw
    assert Brows % (8 * nw) == 0 and b_per_w % ch == 0 and ch <= 128
    mesh = plsc.VectorSubcoreMesh(core_axis_name="c", subcore_axis_name="s")

    @functools.partial(
        pl.kernel,
        out_type=jax.ShapeDtypeStruct((Brows, Dcols), jnp.float32),
        mesh=mesh,
        scratch_types=[
            pltpu.VMEM((b_per_w,), jnp.int32),
            pltpu.VMEM((ch, Dcols), jnp.float32),
            pltpu.SemaphoreType.DMA,
        ],
    )
    def gather(table_hbm, idx_hbm, out_hbm, idx_v, rows_v, sem):
        wid = lax.axis_index("s") * 2 + lax.axis_index("c")
        base = wid * b_per_w
        pltpu.sync_copy(idx_hbm.at[pl.ds(base, b_per_w)], idx_v)

        def body(j, carry):
            pltpu.async_copy(
                table_hbm.at[idx_v.at[pl.ds(j * ch, ch)]], rows_v, sem).wait()
            pltpu.sync_copy(rows_v, out_hbm.at[pl.ds(base + j * ch, ch)])
            return carry

        lax.fori_loop(0, b_per_w // ch, body, 0)

    return gather


def _ffn_body(eot_ref, valid_ref, xs_ref, w1_ref, b1_ref, w2_ref, b2_ref,
              out_ref):
    del eot_ref

    @pl.when(valid_ref[pl.program_id(0)] == 1)
    def _():
        x = xs_ref[...]
        h = (jnp.dot(x, w1_ref[0], preferred_element_type=jnp.float32,
                     precision=lax.Precision.DEFAULT)
             + b1_ref[0])
        h = h * jax.nn.sigmoid(h)
        out_ref[...] = (
            jnp.dot(h, w2_ref[0], preferred_element_type=jnp.float32,
                    precision=lax.Precision.DEFAULT)
            + b2_ref[0])


def _ffn(xs, w1, b1, w2, b2, eot, valid):
    grid_spec = pltpu.PrefetchScalarGridSpec(
        num_scalar_prefetch=2,
        grid=(_NT,),
        in_specs=[
            pl.BlockSpec((_TM, _D), lambda i, eot, v: (i, 0)),
            pl.BlockSpec((1, _D, 2 * _D), lambda i, eot, v: (eot[i], 0, 0)),
            pl.BlockSpec((1, 1, 2 * _D), lambda i, eot, v: (eot[i], 0, 0)),
            pl.BlockSpec((1, 2 * _D, _D), lambda i, eot, v: (eot[i], 0, 0)),
            pl.BlockSpec((1, 1, _D), lambda i, eot, v: (eot[i], 0, 0)),
        ],
        out_specs=pl.BlockSpec((_TM, _D), lambda i, eot, v: (i, 0)),
    )
    return pl.pallas_call(
        _ffn_body,
        grid_spec=grid_spec,
        out_shape=jax.ShapeDtypeStruct((_NPAD, _D), jnp.float32),
    )(eot, valid, xs, w1, b1.reshape(_E, 1, 2 * _D), w2,
      b2.reshape(_E, 1, _D))


def _combine_body(ga_ref, gb_ref, tp_ref, y_ref):
    y_ref[...] = (tp_ref[:, 0:1] * ga_ref[...]
                  + tp_ref[:, 1:2] * gb_ref[...])


def _combine(gpair, tp):
    nblk = _N // _TT
    return pl.pallas_call(
        _combine_body,
        grid=(nblk,),
        in_specs=[
            pl.BlockSpec((_TT, _D), lambda i: (i, 0)),
            pl.BlockSpec((_TT, _D), lambda i: (i + _N // _TT, 0)),
            pl.BlockSpec((_TT, _K), lambda i: (i, 0)),
        ],
        out_specs=pl.BlockSpec((_TT, _D), lambda i: (i, 0)),
        out_shape=jax.ShapeDtypeStruct((_N, _D), jnp.float32),
    )(gpair, gpair, tp)


def kernel(x, gate_w, gate_b, w1, b1, w2, b2):
    x = jnp.asarray(x, dtype=jnp.float32)
    xf = x.reshape(_N, _D)
    noise = (jax.random.normal(jax.random.key(42), (_B, _S, _E)) * 0.01
             ).reshape(_N, _E)

    xn, ti, tp, _us, aux = _gating(xf, gate_w, gate_b, noise)
    tok_of_pos, pos, eot, valid = _dispatch_meta(ti)

    xs = _make_sc_gather(_N, _NPAD, _D, 64)(xn, tok_of_pos)
    out2 = _ffn(xs, w1, b1, w2, b2, eot, valid)
    gpair = _make_sc_gather(_NPAD, _A, _D, 64)(out2, pos.astype(jnp.int32))
    y = _combine(gpair, tp)

    return (y.reshape(_B, _S, _D),
            ti.reshape(_B, _S, _K),
            aux[0, 0])
